# baseline (speedup 1.0000x reference)
import functools

import jax
import jax.numpy as jnp
from jax import lax
from jax.experimental import pallas as pl
from jax.experimental.pallas import tpu as pltpu
from jax.experimental.pallas import tpu_sc as plsc

N = 10000
E = 320000
H = 64
D_OUT = 40

NC = 2
NS = 16
CH = 128
NSLOT = 8

N_PAD = 10240
EPT = 10240
E_PAD = EPT * NC * NS
NCH = EPT // CH
RPT = N_PAD // NS

NF = N * H // 128
NF_PAD = N_PAD * H // 128

_mesh = plsc.VectorSubcoreMesh(core_axis_name="c", subcore_axis_name="s")
_sc_params = pltpu.CompilerParams(use_tc_tiling_on_sc=False)


def _deg_body(eidx_hbm, out_hbm, idx_v, ones_v, zero_v, shared, isem, ssem):
    c = lax.axis_index("c")
    s = lax.axis_index("s")
    wid = c * NS + s

    row0 = wid * NCH
    pltpu.async_copy(eidx_hbm.at[1, pl.ds(row0, NCH)], idx_v, isem)

    def _init(i, _):
        ones_v[pl.ds(i * 16, 16)] = jnp.ones((16,), jnp.float32)
        return 0

    lax.fori_loop(0, CH // 16, _init, 0)

    def _zero(i, _):
        zero_v[pl.ds(i * 16, 16)] = jnp.zeros((16,), jnp.float32)
        return 0

    lax.fori_loop(0, RPT // 16, _zero, 0)
    pltpu.sync_copy(zero_v, shared.at[pl.ds(s * RPT, RPT)])
    pltpu.make_async_copy(eidx_hbm.at[1, pl.ds(row0, NCH)], idx_v,
                          isem).wait()
    plsc.subcore_barrier()

    def _group(g, _):
        for k in range(8):
            pltpu.async_copy(ones_v, shared.at[idx_v.at[g * 8 + k]], ssem,
                             add=True)
        for k in range(8):
            pltpu.make_async_copy(ones_v, shared.at[idx_v.at[0]],
                                  ssem).wait()
        return 0

    lax.fori_loop(0, NCH // 8, _group, 0)
    plsc.subcore_barrier()
    pltpu.sync_copy(shared.at[pl.ds(s * RPT, RPT)],
                    out_hbm.at[pl.ds(c * N_PAD + s * RPT, RPT)])


_deg_call = pl.kernel(
    _deg_body,
    out_type=jax.ShapeDtypeStruct((NC * N_PAD,), jnp.float32),
    mesh=_mesh,
    compiler_params=_sc_params,
    scratch_types=[
        pltpu.VMEM((NCH, CH), jnp.int32),
        pltpu.VMEM((CH,), jnp.float32),
        pltpu.VMEM((RPT,), jnp.float32),
        pltpu.VMEM_SHARED((N_PAD,), jnp.float32),
        pltpu.SemaphoreType.DMA,
        pltpu.SemaphoreType.DMA,
    ],
)


def _spmm_body(F, y_hbm, eidx_hbm, out_hbm, sidx, didx, rows, shared,
               *sems):
    c = lax.axis_index("c")
    s = lax.axis_index("s")
    wid = c * NS + s

    row0 = wid * NCH
    pltpu.async_copy(eidx_hbm.at[0, pl.ds(row0, NCH)], sidx, sems[0])
    pltpu.async_copy(eidx_hbm.at[1, pl.ds(row0, NCH)], didx, sems[1])

    def _zero(i, _):
        for j in range(F // 16):
            rows[i, pl.ds(j * 16, 16)] = jnp.zeros((16,), jnp.float32)
        return 0

    lax.fori_loop(0, CH, _zero, 0)
    for r in range(RPT // CH):
        pltpu.sync_copy(rows.at[pl.ds(0, CH)],
                        shared.at[pl.ds(s * RPT + r * CH, CH)])
    pltpu.make_async_copy(eidx_hbm.at[0, pl.ds(row0, NCH)], sidx,
                          sems[0]).wait()
    pltpu.make_async_copy(eidx_hbm.at[1, pl.ds(row0, NCH)], didx,
                          sems[1]).wait()
    plsc.subcore_barrier()

    def _slot(k):
        return rows.at[pl.ds(k * CH, CH)]

    for k in range(NSLOT):
        pltpu.async_copy(y_hbm.at[sidx.at[k]], _slot(k), sems[k])

    def _step(m, _):
        for k in range(NSLOT):
            ch = m * NSLOT + k
            pltpu.make_async_copy(y_hbm.at[sidx.at[0]], _slot(k),
                                  sems[k]).wait()
            pltpu.sync_copy(_slot(k), shared.at[didx.at[ch]], add=True)

            @pl.when(m < NCH // NSLOT - 1)
            def _():
                pltpu.async_copy(y_hbm.at[sidx.at[ch + NSLOT]], _slot(k),
                                 sems[k])
        return 0

    lax.fori_loop(0, NCH // NSLOT, _step, 0)
    plsc.subcore_barrier()
    pltpu.sync_copy(shared.at[pl.ds(s * RPT, RPT)],
                    out_hbm.at[c, pl.ds(s * RPT, RPT)])


def _make_spmm(F):
    return pl.kernel(
        functools.partial(_spmm_body, F),
        out_type=jax.ShapeDtypeStruct((NC, N_PAD, F), jnp.float32),
        mesh=_mesh,
        compiler_params=_sc_params,
        scratch_types=[
            pltpu.VMEM((NCH, CH), jnp.int32),
            pltpu.VMEM((NCH, CH), jnp.int32),
            pltpu.VMEM((NSLOT * CH, F), jnp.float32),
            pltpu.VMEM_SHARED((N_PAD, F), jnp.float32),
        ] + [pltpu.SemaphoreType.DMA] * NSLOT,
    )


_spmm64 = _make_spmm(H)


def _bdiag(w):
    k, f = w.shape
    z = jnp.zeros_like(w)
    return jnp.concatenate([jnp.concatenate([w, z], 1),
                            jnp.concatenate([z, w], 1)], 0)


def _mm1_body(xf_ref, w_ref, d_ref, y_ref):
    y_ref[...] = d_ref[...] * jnp.dot(xf_ref[...], w_ref[...],
                                      preferred_element_type=jnp.float32)


def _mm1(xf, wbd, dinvf):
    bm = 1000
    return pl.pallas_call(
        _mm1_body,
        grid=(NF // bm,),
        in_specs=[pl.BlockSpec((bm, 256), lambda i: (i, 0)),
                  pl.BlockSpec((256, 128), lambda i: (0, 0)),
                  pl.BlockSpec((bm, 128), lambda i: (i, 0))],
        out_specs=pl.BlockSpec((bm, 128), lambda i: (i, 0)),
        out_shape=jax.ShapeDtypeStruct((NF_PAD, 128), jnp.float32),
    )(xf, wbd, dinvf)


def _layer_body(z0_ref, z1_ref, y_ref, d_ref, b_ref, w_ref, xl_ref, yn_ref):
    d = d_ref[...]
    xl = jnp.maximum(d * (z0_ref[...] + z1_ref[...] + y_ref[...])
                     + b_ref[...], 0.0)
    xl_ref[...] = xl
    yn_ref[...] = d * jnp.dot(xl, w_ref[...],
                              preferred_element_type=jnp.float32)


def _layer(zf, yf, dinvf, bf, wbd):
    bm = 512
    nb = NF_PAD // bm
    return pl.pallas_call(
        _layer_body,
        grid=(nb,),
        in_specs=[pl.BlockSpec((bm, 128), lambda i: (i, 0)),
                  pl.BlockSpec((bm, 128), lambda i: (i + nb, 0)),
                  pl.BlockSpec((bm, 128), lambda i: (i, 0)),
                  pl.BlockSpec((bm, 128), lambda i: (i, 0)),
                  pl.BlockSpec((1, 128), lambda i: (0, 0)),
                  pl.BlockSpec((128, 128), lambda i: (0, 0))],
        out_specs=[pl.BlockSpec((bm, 128), lambda i: (i, 0)),
                   pl.BlockSpec((bm, 128), lambda i: (i, 0))],
        out_shape=[jax.ShapeDtypeStruct((NF, 128), jnp.float32),
                   jax.ShapeDtypeStruct((NF_PAD, 128), jnp.float32)],
    )(zf, zf, yf, dinvf, bf, wbd)


def _layer3_body(z0_ref, z1_ref, y_ref, d_ref, b_ref, x1_ref, wa_ref, wb_ref,
                 x2_ref, yn_ref):
    d = d_ref[...]
    x2 = jnp.maximum(d * (z0_ref[...] + z1_ref[...] + y_ref[...])
                     + b_ref[...], 0.0)
    x2_ref[...] = x2
    yn_ref[...] = d * (jnp.dot(x1_ref[...], wa_ref[...],
                               preferred_element_type=jnp.float32)
                       + jnp.dot(x2, wb_ref[...],
                                 preferred_element_type=jnp.float32))


def _layer3(zf, yf, dinvf, bf, x1f, wbda, wbdb):
    bm = 512
    nb = NF_PAD // bm
    return pl.pallas_call(
        _layer3_body,
        grid=(nb,),
        in_specs=[pl.BlockSpec((bm, 128), lambda i: (i, 0)),
                  pl.BlockSpec((bm, 128), lambda i: (i + nb, 0)),
                  pl.BlockSpec((bm, 128), lambda i: (i, 0)),
                  pl.BlockSpec((bm, 128), lambda i: (i, 0)),
                  pl.BlockSpec((1, 128), lambda i: (0, 0)),
                  pl.BlockSpec((bm, 128), lambda i: (i, 0)),
                  pl.BlockSpec((128, 128), lambda i: (0, 0)),
                  pl.BlockSpec((128, 128), lambda i: (0, 0))],
        out_specs=[pl.BlockSpec((bm, 128), lambda i: (i, 0)),
                   pl.BlockSpec((bm, 128), lambda i: (i, 0))],
        out_shape=[jax.ShapeDtypeStruct((NF, 128), jnp.float32),
                   jax.ShapeDtypeStruct((NF_PAD, 128), jnp.float32)],
    )(zf, zf, yf, dinvf, bf, x1f, wbda, wbdb)


def _final_body(z0_ref, z1_ref, y_ref, d_ref, b_ref, o_ref):
    o_ref[...] = (d_ref[...] * (z0_ref[...] + z1_ref[...] + y_ref[...])
                  + b_ref[...])


def _final(zf, yf, dinvf, bf):
    bm = 512
    nb = NF_PAD // bm
    return pl.pallas_call(
        _final_body,
        grid=(nb,),
        in_specs=[pl.BlockSpec((bm, 128), lambda i: (i, 0)),
                  pl.BlockSpec((bm, 128), lambda i: (i + nb, 0)),
                  pl.BlockSpec((bm, 128), lambda i: (i, 0)),
                  pl.BlockSpec((bm, 128), lambda i: (i, 0)),
                  pl.BlockSpec((1, 128), lambda i: (0, 0))],
        out_specs=pl.BlockSpec((bm, 128), lambda i: (i, 0)),
        out_shape=jax.ShapeDtypeStruct((NF, 128), jnp.float32),
    )(zf, zf, yf, dinvf, bf)


def kernel(x, edge_index, percent, ricci_curvature, W1, b1, W2, b2, W3, b3):
    del percent, ricci_curvature
    fill = N + (jnp.arange(E_PAD - E, dtype=jnp.int32) % (N_PAD - N))
    eidx = jnp.concatenate(
        [edge_index.reshape(2, E // CH, CH),
         jnp.broadcast_to(fill.reshape(1, (E_PAD - E) // CH, CH),
                          (2, (E_PAD - E) // CH, CH))], axis=1)

    degp = _deg_call(eidx)
    degf = degp.reshape(2 * N_PAD // 128, 128)
    nh = N_PAD // 128
    dinvg = (degf[:nh] + degf[nh:] + 1.0) ** -0.5
    dinvf = jnp.repeat(dinvg.reshape(N_PAD)[:N], H).reshape(NF, 128)
    xf = x.reshape(NF, 256)

    y1f = _mm1(xf, _bdiag(W1), dinvf)
    z1 = _spmm64(y1f.reshape(N_PAD, H), eidx)
    x1f, y2f = _layer(z1.reshape(2 * NF_PAD, 128), y1f, dinvf,
                      jnp.tile(b1, 2)[None], _bdiag(W2))

    z2 = _spmm64(y2f.reshape(N_PAD, H), eidx)
    w3p = jnp.pad(W3, ((0, 0), (0, H - D_OUT)))
    x2f, y3f = _layer3(z2.reshape(2 * NF_PAD, 128), y2f, dinvf,
                       jnp.tile(b2, 2)[None], x1f,
                       _bdiag(w3p[:H]), _bdiag(w3p[H:]))

    z3 = _spmm64(y3f.reshape(N_PAD, H), eidx)
    b3f = jnp.tile(jnp.pad(b3, (0, H - D_OUT)), 2)[None]
    vf = _final(z3.reshape(2 * NF_PAD, 128), y3f, dinvf, b3f)
    out = vf.reshape(N, H)[:, :D_OUT]
    x1 = x1f.reshape(N, H)
    x2 = x2f.reshape(N, H)
    return (out, x1, x2)


# speedup vs baseline: 1.0052x; 1.0052x over previous
import functools

import jax
import jax.numpy as jnp
from jax import lax
from jax.experimental import pallas as pl
from jax.experimental.pallas import tpu as pltpu
from jax.experimental.pallas import tpu_sc as plsc

N = 10000
E = 320000
H = 64
D_OUT = 40

NC = 2
NS = 16
CH = 128
NSLOT = 4

N_PAD = 10240
EPT = 10240
E_PAD = EPT * NC * NS
NCH = EPT // CH
RPT = N_PAD // NS

NF = N * H // 128
NF_PAD = N_PAD * H // 128

_mesh = plsc.VectorSubcoreMesh(core_axis_name="c", subcore_axis_name="s")
_sc_params = pltpu.CompilerParams(use_tc_tiling_on_sc=False)


def _deg_body(eidx_hbm, out_hbm, idx_v, ones_v, zero_v, shared, isem, ssem):
    c = lax.axis_index("c")
    s = lax.axis_index("s")
    wid = c * NS + s

    row0 = wid * NCH
    pltpu.async_copy(eidx_hbm.at[1, pl.ds(row0, NCH)], idx_v, isem)

    def _init(i, _):
        ones_v[pl.ds(i * 16, 16)] = jnp.ones((16,), jnp.float32)
        return 0

    lax.fori_loop(0, CH // 16, _init, 0)

    def _zero(i, _):
        zero_v[pl.ds(i * 16, 16)] = jnp.zeros((16,), jnp.float32)
        return 0

    lax.fori_loop(0, RPT // 16, _zero, 0)
    pltpu.sync_copy(zero_v, shared.at[pl.ds(s * RPT, RPT)])
    pltpu.make_async_copy(eidx_hbm.at[1, pl.ds(row0, NCH)], idx_v,
                          isem).wait()
    plsc.subcore_barrier()

    def _group(g, _):
        for k in range(8):
            pltpu.async_copy(ones_v, shared.at[idx_v.at[g * 8 + k]], ssem,
                             add=True)
        for k in range(8):
            pltpu.make_async_copy(ones_v, shared.at[idx_v.at[0]],
                                  ssem).wait()
        return 0

    lax.fori_loop(0, NCH // 8, _group, 0)
    plsc.subcore_barrier()
    pltpu.sync_copy(shared.at[pl.ds(s * RPT, RPT)],
                    out_hbm.at[pl.ds(c * N_PAD + s * RPT, RPT)])


_deg_call = pl.kernel(
    _deg_body,
    out_type=jax.ShapeDtypeStruct((NC * N_PAD,), jnp.float32),
    mesh=_mesh,
    compiler_params=_sc_params,
    scratch_types=[
        pltpu.VMEM((NCH, CH), jnp.int32),
        pltpu.VMEM((CH,), jnp.float32),
        pltpu.VMEM((RPT,), jnp.float32),
        pltpu.VMEM_SHARED((N_PAD,), jnp.float32),
        pltpu.SemaphoreType.DMA,
        pltpu.SemaphoreType.DMA,
    ],
)


def _spmm_body(F, y_hbm, eidx_hbm, out_hbm, sidx, didx, rows, shared,
               *sems):
    c = lax.axis_index("c")
    s = lax.axis_index("s")
    wid = c * NS + s

    row0 = wid * NCH
    pltpu.async_copy(eidx_hbm.at[0, pl.ds(row0, NCH)], sidx, sems[0])
    pltpu.async_copy(eidx_hbm.at[1, pl.ds(row0, NCH)], didx, sems[1])

    def _zero(i, _):
        for j in range(F // 16):
            rows[i, pl.ds(j * 16, 16)] = jnp.zeros((16,), jnp.float32)
        return 0

    lax.fori_loop(0, CH, _zero, 0)
    for r in range(RPT // CH):
        pltpu.sync_copy(rows.at[pl.ds(0, CH)],
                        shared.at[pl.ds(s * RPT + r * CH, CH)])
    pltpu.make_async_copy(eidx_hbm.at[0, pl.ds(row0, NCH)], sidx,
                          sems[0]).wait()
    pltpu.make_async_copy(eidx_hbm.at[1, pl.ds(row0, NCH)], didx,
                          sems[1]).wait()
    plsc.subcore_barrier()

    def _slot(k):
        return rows.at[pl.ds(k * CH, CH)]

    for k in range(NSLOT):
        pltpu.async_copy(y_hbm.at[sidx.at[k]], _slot(k), sems[k])

    def _step(m, _):
        for k in range(NSLOT):
            ch = m * NSLOT + k
            pltpu.make_async_copy(y_hbm.at[sidx.at[0]], _slot(k),
                                  sems[k]).wait()
            pltpu.sync_copy(_slot(k), shared.at[didx.at[ch]], add=True)

            @pl.when(m < NCH // NSLOT - 1)
            def _():
                pltpu.async_copy(y_hbm.at[sidx.at[ch + NSLOT]], _slot(k),
                                 sems[k])
        return 0

    lax.fori_loop(0, NCH // NSLOT, _step, 0)
    plsc.subcore_barrier()
    pltpu.sync_copy(shared.at[pl.ds(s * RPT, RPT)],
                    out_hbm.at[c, pl.ds(s * RPT, RPT)])


def _make_spmm(F):
    return pl.kernel(
        functools.partial(_spmm_body, F),
        out_type=jax.ShapeDtypeStruct((NC, N_PAD, F), jnp.float32),
        mesh=_mesh,
        compiler_params=_sc_params,
        scratch_types=[
            pltpu.VMEM((NCH, CH), jnp.int32),
            pltpu.VMEM((NCH, CH), jnp.int32),
            pltpu.VMEM((NSLOT * CH, F), jnp.float32),
            pltpu.VMEM_SHARED((N_PAD, F), jnp.float32),
        ] + [pltpu.SemaphoreType.DMA] * NSLOT,
    )


_spmm64 = _make_spmm(H)


def _bdiag(w):
    k, f = w.shape
    z = jnp.zeros_like(w)
    return jnp.concatenate([jnp.concatenate([w, z], 1),
                            jnp.concatenate([z, w], 1)], 0)


def _mm1_body(xf_ref, w_ref, d_ref, y_ref):
    y_ref[...] = d_ref[...] * jnp.dot(xf_ref[...], w_ref[...],
                                      preferred_element_type=jnp.float32)


def _mm1(xf, wbd, dinvf):
    bm = 1000
    return pl.pallas_call(
        _mm1_body,
        grid=(NF // bm,),
        in_specs=[pl.BlockSpec((bm, 256), lambda i: (i, 0)),
                  pl.BlockSpec((256, 128), lambda i: (0, 0)),
                  pl.BlockSpec((bm, 128), lambda i: (i, 0))],
        out_specs=pl.BlockSpec((bm, 128), lambda i: (i, 0)),
        out_shape=jax.ShapeDtypeStruct((NF_PAD, 128), jnp.float32),
    )(xf, wbd, dinvf)


def _layer_body(z0_ref, z1_ref, y_ref, d_ref, b_ref, w_ref, xl_ref, yn_ref):
    d = d_ref[...]
    xl = jnp.maximum(d * (z0_ref[...] + z1_ref[...] + y_ref[...])
                     + b_ref[...], 0.0)
    xl_ref[...] = xl
    yn_ref[...] = d * jnp.dot(xl, w_ref[...],
                              preferred_element_type=jnp.float32)


def _layer(zf, yf, dinvf, bf, wbd):
    bm = 512
    nb = NF_PAD // bm
    return pl.pallas_call(
        _layer_body,
        grid=(nb,),
        in_specs=[pl.BlockSpec((bm, 128), lambda i: (i, 0)),
                  pl.BlockSpec((bm, 128), lambda i: (i + nb, 0)),
                  pl.BlockSpec((bm, 128), lambda i: (i, 0)),
                  pl.BlockSpec((bm, 128), lambda i: (i, 0)),
                  pl.BlockSpec((1, 128), lambda i: (0, 0)),
                  pl.BlockSpec((128, 128), lambda i: (0, 0))],
        out_specs=[pl.BlockSpec((bm, 128), lambda i: (i, 0)),
                   pl.BlockSpec((bm, 128), lambda i: (i, 0))],
        out_shape=[jax.ShapeDtypeStruct((NF, 128), jnp.float32),
                   jax.ShapeDtypeStruct((NF_PAD, 128), jnp.float32)],
    )(zf, zf, yf, dinvf, bf, wbd)


def _layer3_body(z0_ref, z1_ref, y_ref, d_ref, b_ref, x1_ref, wa_ref, wb_ref,
                 x2_ref, yn_ref):
    d = d_ref[...]
    x2 = jnp.maximum(d * (z0_ref[...] + z1_ref[...] + y_ref[...])
                     + b_ref[...], 0.0)
    x2_ref[...] = x2
    yn_ref[...] = d * (jnp.dot(x1_ref[...], wa_ref[...],
                               preferred_element_type=jnp.float32)
                       + jnp.dot(x2, wb_ref[...],
                                 preferred_element_type=jnp.float32))


def _layer3(zf, yf, dinvf, bf, x1f, wbda, wbdb):
    bm = 512
    nb = NF_PAD // bm
    return pl.pallas_call(
        _layer3_body,
        grid=(nb,),
        in_specs=[pl.BlockSpec((bm, 128), lambda i: (i, 0)),
                  pl.BlockSpec((bm, 128), lambda i: (i + nb, 0)),
                  pl.BlockSpec((bm, 128), lambda i: (i, 0)),
                  pl.BlockSpec((bm, 128), lambda i: (i, 0)),
                  pl.BlockSpec((1, 128), lambda i: (0, 0)),
                  pl.BlockSpec((bm, 128), lambda i: (i, 0)),
                  pl.BlockSpec((128, 128), lambda i: (0, 0)),
                  pl.BlockSpec((128, 128), lambda i: (0, 0))],
        out_specs=[pl.BlockSpec((bm, 128), lambda i: (i, 0)),
                   pl.BlockSpec((bm, 128), lambda i: (i, 0))],
        out_shape=[jax.ShapeDtypeStruct((NF, 128), jnp.float32),
                   jax.ShapeDtypeStruct((NF_PAD, 128), jnp.float32)],
    )(zf, zf, yf, dinvf, bf, x1f, wbda, wbdb)


def _final_body(z0_ref, z1_ref, y_ref, d_ref, b_ref, o_ref):
    o_ref[...] = (d_ref[...] * (z0_ref[...] + z1_ref[...] + y_ref[...])
                  + b_ref[...])


def _final(zf, yf, dinvf, bf):
    bm = 512
    nb = NF_PAD // bm
    return pl.pallas_call(
        _final_body,
        grid=(nb,),
        in_specs=[pl.BlockSpec((bm, 128), lambda i: (i, 0)),
                  pl.BlockSpec((bm, 128), lambda i: (i + nb, 0)),
                  pl.BlockSpec((bm, 128), lambda i: (i, 0)),
                  pl.BlockSpec((bm, 128), lambda i: (i, 0)),
                  pl.BlockSpec((1, 128), lambda i: (0, 0))],
        out_specs=pl.BlockSpec((bm, 128), lambda i: (i, 0)),
        out_shape=jax.ShapeDtypeStruct((NF, 128), jnp.float32),
    )(zf, zf, yf, dinvf, bf)


def kernel(x, edge_index, percent, ricci_curvature, W1, b1, W2, b2, W3, b3):
    del percent, ricci_curvature
    fill = N + (jnp.arange(E_PAD - E, dtype=jnp.int32) % (N_PAD - N))
    eidx = jnp.concatenate(
        [edge_index.reshape(2, E // CH, CH),
         jnp.broadcast_to(fill.reshape(1, (E_PAD - E) // CH, CH),
                          (2, (E_PAD - E) // CH, CH))], axis=1)

    degp = _deg_call(eidx)
    degf = degp.reshape(2 * N_PAD // 128, 128)
    nh = N_PAD // 128
    dinvg = (degf[:nh] + degf[nh:] + 1.0) ** -0.5
    dinvf = jnp.repeat(dinvg.reshape(N_PAD)[:N], H).reshape(NF, 128)
    xf = x.reshape(NF, 256)

    y1f = _mm1(xf, _bdiag(W1), dinvf)
    z1 = _spmm64(y1f.reshape(N_PAD, H), eidx)
    x1f, y2f = _layer(z1.reshape(2 * NF_PAD, 128), y1f, dinvf,
                      jnp.tile(b1, 2)[None], _bdiag(W2))

    z2 = _spmm64(y2f.reshape(N_PAD, H), eidx)
    w3p = jnp.pad(W3, ((0, 0), (0, H - D_OUT)))
    x2f, y3f = _layer3(z2.reshape(2 * NF_PAD, 128), y2f, dinvf,
                       jnp.tile(b2, 2)[None], x1f,
                       _bdiag(w3p[:H]), _bdiag(w3p[H:]))

    z3 = _spmm64(y3f.reshape(N_PAD, H), eidx)
    b3f = jnp.tile(jnp.pad(b3, (0, H - D_OUT)), 2)[None]
    vf = _final(z3.reshape(2 * NF_PAD, 128), y3f, dinvf, b3f)
    out = vf.reshape(N, H)[:, :D_OUT]
    x1 = x1f.reshape(N, H)
    x2 = x2f.reshape(N, H)
    return (out, x1, x2)
